# baseline (speedup 1.0000x reference)
import functools

import jax
import jax.numpy as jnp
from jax import lax
from jax.experimental import pallas as pl
from jax.experimental.pallas import tpu as pltpu
from jax.experimental.pallas import tpu_sc as plsc

_NUM_CORES = 2
_NUM_SUBCORES = 16
_NUM_WORKERS = _NUM_CORES * _NUM_SUBCORES
_LANES = 16
_SLAB_ROWS = 32


@functools.cache
def _build_scatter(n_rows: int, n_cols: int, nnz_per_row: int):
    n_wrows = n_rows // 2
    rows_per_worker = n_rows // _NUM_WORKERS
    slabs_per_worker = rows_per_worker // _SLAB_ROWS
    slab_nnz = _SLAB_ROWS * nnz_per_row
    stage_words = ((slab_nnz + 127) // 128) * 128

    mesh = plsc.VectorSubcoreMesh(
        core_axis_name="c", subcore_axis_name="s",
        num_cores=_NUM_CORES, num_subcores=_NUM_SUBCORES)

    @functools.partial(
        pl.kernel,
        out_type=jax.ShapeDtypeStruct((n_wrows, n_cols), jnp.int32),
        mesh=mesh,
        compiler_params=pltpu.CompilerParams(needs_layout_passes=False),
        scratch_types=[
            pltpu.VMEM((stage_words,), jnp.float32),
            pltpu.VMEM((stage_words,), jnp.int32),
            pltpu.VMEM((_SLAB_ROWS // 2, n_cols), jnp.int32),
        ],
    )
    def scatter_kernel(c0_hbm, c1_hbm, w_hbm, vals_v, cols_v, buf):
        wid = lax.axis_index("s") * _NUM_CORES + lax.axis_index("c")

        iot = lax.iota(jnp.int32, _LANES)
        zzi = jnp.zeros((_LANES,), jnp.int32)

        def _zero(i, carry):
            for r in range(_SLAB_ROWS // 2):
                buf[r, pl.ds(i * _LANES, _LANES)] = zzi
            return carry
        lax.fori_loop(0, n_cols // _LANES, _zero, 0)

        def _slab(c, carry):
            wr0 = wid * (rows_per_worker // 2) + c * (_SLAB_ROWS // 2)
            off = wr0 * (2 * nnz_per_row)
            pltpu.sync_copy(c0_hbm.at[pl.ds(off, slab_nnz)],
                            vals_v.at[pl.ds(0, slab_nnz)])
            pltpu.sync_copy(c1_hbm.at[pl.ds(off, slab_nnz)],
                            cols_v.at[pl.ds(0, slab_nnz)])
            zzf = jnp.zeros((_LANES,), jnp.float32)
            for parity in (0, 1):
                shift = 16 * parity
                gbase = iot * (2 * nnz_per_row) + parity * nnz_per_row
                colp = plsc.load_gather(cols_v, [gbase])
                valp = plsc.load_gather(vals_v, [gbase])
                for k in range(1, nnz_per_row + 1):
                    if k < nnz_per_row:
                        colk = plsc.load_gather(cols_v, [gbase + k])
                        valk = plsc.load_gather(vals_v, [gbase + k])
                        dup = colp == colk
                        valk = valk + jnp.where(dup, valp, zzf)
                        v_out = jnp.where(dup, zzf, valp)
                    else:
                        v_out = valp
                    nb = lax.shift_right_logical(
                        plsc.bitcast(v_out, jnp.int32) + 0x8000, 16)
                    plsc.addupdate_scatter(
                        buf, [iot, colp], lax.shift_left(nb, shift))
                    if k < nnz_per_row:
                        colp, valp = colk, valk
            pltpu.sync_copy(buf, w_hbm.at[pl.ds(wr0, _SLAB_ROWS // 2)])
            for parity in (0, 1):
                gbase = iot * (2 * nnz_per_row) + parity * nnz_per_row
                for k in range(nnz_per_row):
                    colk = plsc.load_gather(cols_v, [gbase + k])
                    plsc.store_scatter(buf, [iot, colk], zzi)
            return carry
        lax.fori_loop(0, slabs_per_worker, _slab, 0)

    return scatter_kernel


@functools.cache
def _build_cast(m: int, k: int):
    def cast_body(x_ref, o_ref):
        o_ref[...] = x_ref[...].astype(jnp.bfloat16)

    return pl.pallas_call(
        cast_body,
        out_shape=jax.ShapeDtypeStruct((m, k), jnp.bfloat16),
    )


@functools.cache
def _build_matmul(m: int, k: int, n: int):
    bn = 1024

    def mm_body(x_ref, w_ref, o_ref):
        w = w_ref.bitcast(jnp.bfloat16)[...]
        o_ref[...] = lax.dot_general(
            x_ref[...], w,
            dimension_numbers=(((1,), (1,)), ((), ())),
            preferred_element_type=jnp.float32)

    return pl.pallas_call(
        mm_body,
        grid=(n // bn,) if n > bn else (1,),
        in_specs=[
            pl.BlockSpec((m, k), lambda i: (0, 0)),
            pl.BlockSpec((min(bn, n) // 2, k), lambda i: (i, 0)),
        ],
        out_specs=pl.BlockSpec((m, min(bn, n)), lambda i: (0, i)),
        out_shape=jax.ShapeDtypeStruct((m, n), jnp.float32),
    )


_N_CHUNKS = 4


def kernel(x, c_0, c_1, c_2, c_3, c_4):
    original_shape = x.shape
    n_cols = original_shape[-1]
    n_rows = c_2.shape[0] - 1
    nnz_per_row = c_0.shape[0] // n_rows
    x_flat = x.reshape(-1, n_cols)

    rows_c = n_rows // _N_CHUNKS
    nnz_c = rows_c * nnz_per_row
    scatter = _build_scatter(rows_c, n_cols, nnz_per_row)
    matmul = _build_matmul(x_flat.shape[0], n_cols, rows_c)
    ws = [scatter(lax.dynamic_slice_in_dim(c_0, c * nnz_c, nnz_c),
                  lax.dynamic_slice_in_dim(c_1, c * nnz_c, nnz_c))
          for c in range(_N_CHUNKS)]
    x_bf = _build_cast(x_flat.shape[0], n_cols)(x_flat)
    outs = [matmul(x_bf, w_c) for w_c in ws]
    out_flat = outs[0] if _N_CHUNKS == 1 else jnp.concatenate(outs, axis=1)
    return out_flat.reshape(*original_shape[:-1], n_rows)


# speedup vs baseline: 1.2899x; 1.2899x over previous
import functools

import jax
import jax.numpy as jnp
from jax import lax
from jax.experimental import pallas as pl
from jax.experimental.pallas import tpu as pltpu
from jax.experimental.pallas import tpu_sc as plsc

_NUM_CORES = 2
_NUM_SUBCORES = 16
_NUM_WORKERS = _NUM_CORES * _NUM_SUBCORES
_LANES = 16
_SLAB_ROWS = 32


@functools.cache
def _build_scatter(n_rows: int, n_cols: int, nnz_per_row: int):
    n_wrows = n_rows // 2
    rows_per_worker = n_rows // _NUM_WORKERS
    slabs_per_worker = rows_per_worker // _SLAB_ROWS
    slab_nnz = _SLAB_ROWS * nnz_per_row
    stage_words = ((slab_nnz + 127) // 128) * 128

    mesh = plsc.VectorSubcoreMesh(
        core_axis_name="c", subcore_axis_name="s",
        num_cores=_NUM_CORES, num_subcores=_NUM_SUBCORES)

    @functools.partial(
        pl.kernel,
        out_type=jax.ShapeDtypeStruct((n_wrows, n_cols), jnp.int32),
        mesh=mesh,
        compiler_params=pltpu.CompilerParams(needs_layout_passes=False),
        scratch_types=[
            pltpu.VMEM((stage_words,), jnp.float32),
            pltpu.VMEM((stage_words,), jnp.int32),
            pltpu.VMEM((_SLAB_ROWS // 2, n_cols), jnp.int32),
        ],
    )
    def scatter_kernel(c0_hbm, c1_hbm, w_hbm, vals_v, cols_v, buf):
        wid = lax.axis_index("s") * _NUM_CORES + lax.axis_index("c")

        iot = lax.iota(jnp.int32, _LANES)
        zzi = jnp.zeros((_LANES,), jnp.int32)

        def _zero(i, carry):
            for r in range(_SLAB_ROWS // 2):
                buf[r, pl.ds(i * _LANES, _LANES)] = zzi
            return carry
        lax.fori_loop(0, n_cols // _LANES, _zero, 0)

        def _slab(c, carry):
            wr0 = wid * (rows_per_worker // 2) + c * (_SLAB_ROWS // 2)
            off = wr0 * (2 * nnz_per_row)
            pltpu.sync_copy(c0_hbm.at[pl.ds(off, slab_nnz)],
                            vals_v.at[pl.ds(0, slab_nnz)])
            pltpu.sync_copy(c1_hbm.at[pl.ds(off, slab_nnz)],
                            cols_v.at[pl.ds(0, slab_nnz)])
            zzf = jnp.zeros((_LANES,), jnp.float32)
            for parity in (0, 1):
                shift = 16 * parity
                gbase = iot * (2 * nnz_per_row) + parity * nnz_per_row
                colp = plsc.load_gather(cols_v, [gbase])
                valp = plsc.load_gather(vals_v, [gbase])
                for k in range(1, nnz_per_row + 1):
                    if k < nnz_per_row:
                        colk = plsc.load_gather(cols_v, [gbase + k])
                        valk = plsc.load_gather(vals_v, [gbase + k])
                        dup = colp == colk
                        valk = valk + jnp.where(dup, valp, zzf)
                        v_out = jnp.where(dup, zzf, valp)
                    else:
                        v_out = valp
                    nb = lax.shift_right_logical(
                        plsc.bitcast(v_out, jnp.int32) + 0x8000, 16)
                    plsc.addupdate_scatter(
                        buf, [iot, colp], lax.shift_left(nb, shift))
                    if k < nnz_per_row:
                        colp, valp = colk, valk
            pltpu.sync_copy(buf, w_hbm.at[pl.ds(wr0, _SLAB_ROWS // 2)])
            for parity in (0, 1):
                gbase = iot * (2 * nnz_per_row) + parity * nnz_per_row
                for k in range(nnz_per_row):
                    colk = plsc.load_gather(cols_v, [gbase + k])
                    plsc.store_scatter(buf, [iot, colk], zzi)
            return carry
        lax.fori_loop(0, slabs_per_worker, _slab, 0)

    return scatter_kernel


@functools.cache
def _build_cast(m: int, k: int):
    def cast_body(x_ref, o_ref):
        o_ref[...] = x_ref[...].astype(jnp.bfloat16)

    return pl.pallas_call(
        cast_body,
        out_shape=jax.ShapeDtypeStruct((m, k), jnp.bfloat16),
    )


@functools.cache
def _build_matmul(m: int, k: int, n: int):
    bn = 512

    def mm_body(x_ref, w_ref, o_ref):
        w = w_ref.bitcast(jnp.bfloat16)[...]
        o_ref[...] = lax.dot_general(
            x_ref[...], w,
            dimension_numbers=(((1,), (1,)), ((), ())),
            preferred_element_type=jnp.float32)

    return pl.pallas_call(
        mm_body,
        grid=(n // bn,) if n > bn else (1,),
        in_specs=[
            pl.BlockSpec((m, k), lambda i: (0, 0)),
            pl.BlockSpec((min(bn, n) // 2, k), lambda i: (i, 0)),
        ],
        out_specs=pl.BlockSpec((m, min(bn, n)), lambda i: (0, i)),
        out_shape=jax.ShapeDtypeStruct((m, n), jnp.float32),
    )


_N_CHUNKS = 1


def kernel(x, c_0, c_1, c_2, c_3, c_4):
    original_shape = x.shape
    n_cols = original_shape[-1]
    n_rows = c_2.shape[0] - 1
    nnz_per_row = c_0.shape[0] // n_rows
    x_flat = x.reshape(-1, n_cols)

    rows_c = n_rows // _N_CHUNKS
    nnz_c = rows_c * nnz_per_row
    scatter = _build_scatter(rows_c, n_cols, nnz_per_row)
    matmul = _build_matmul(x_flat.shape[0], n_cols, rows_c)
    ws = [scatter(lax.dynamic_slice_in_dim(c_0, c * nnz_c, nnz_c),
                  lax.dynamic_slice_in_dim(c_1, c * nnz_c, nnz_c))
          for c in range(_N_CHUNKS)]
    x_bf = _build_cast(x_flat.shape[0], n_cols)(x_flat)
    outs = [matmul(x_bf, w_c) for w_c in ws]
    out_flat = outs[0] if _N_CHUNKS == 1 else jnp.concatenate(outs, axis=1)
    return out_flat.reshape(*original_shape[:-1], n_rows)
